# Initial kernel scaffold; baseline (speedup 1.0000x reference)
import jax
import jax.numpy as jnp
from jax.experimental import pallas as pl


def kernel(features, edges, edges2, edge_features, additional_feature, W1_l, b1, W1_r, W2_l, b2, W2_r):
    raise NotImplementedError("write your pallas kernel here")


# speedup vs baseline: 4.1429x; 4.1429x over previous
import functools

import jax
import jax.numpy as jnp
from jax import lax
from jax.experimental import pallas as pl
from jax.experimental.pallas import tpu as pltpu
from jax.experimental.pallas import tpu_sc as plsc

N = 10000
D = 128
E = 320000

NC = 2
NS = 16
NW = NC * NS
C = 128

ROWS_PER_TILE = 640
N_ACC = NS * ROWS_PER_TILE
DUMMY_DST = N
EDGES_PER_TILE = ((E + NW * C - 1) // (NW * C)) * C
E_PAD = EDGES_PER_TILE * NW
CHUNKS_PER_TILE = EDGES_PER_TILE // C

_MESH = plsc.VectorSubcoreMesh(
    core_axis_name="c", subcore_axis_name="s",
    num_cores=NC, num_subcores=NS)


def _make_sc_agg():
  out_type = (jax.ShapeDtypeStruct((NC, N_ACC, D), jnp.float32),)
  scratch = (
      pltpu.VMEM_SHARED((N_ACC, D), jnp.float32),
      pltpu.VMEM((C,), jnp.int32),
      pltpu.VMEM((C,), jnp.int32),
      pltpu.VMEM((C, D), jnp.float32),
      pltpu.SemaphoreType.DMA,
  )

  def body(x_hbm, src_hbm, dst_hbm, zeros_hbm,
           out_sums, acc_sh, sidx, didx, rows, sem):
    cid = lax.axis_index("c")
    sid = lax.axis_index("s")
    wid = cid * NS + sid
    r0 = sid * ROWS_PER_TILE

    pltpu.sync_copy(zeros_hbm.at[pl.ds(r0, ROWS_PER_TILE)],
                    acc_sh.at[pl.ds(r0, ROWS_PER_TILE)])
    plsc.subcore_barrier()

    e0 = wid * EDGES_PER_TILE

    def chunk(t, carry):
      off = pl.multiple_of(e0 + t * C, C)
      pltpu.sync_copy(src_hbm.at[pl.ds(off, C)], sidx)
      pltpu.async_copy(x_hbm.at[sidx], rows, sem).wait()
      pltpu.sync_copy(dst_hbm.at[pl.ds(off, C)], didx)
      pltpu.sync_copy(rows, acc_sh.at[didx], add=True)
      return carry

    lax.fori_loop(0, CHUNKS_PER_TILE, chunk, 0)
    plsc.subcore_barrier()

    pltpu.sync_copy(acc_sh.at[pl.ds(r0, ROWS_PER_TILE)],
                    out_sums.at[cid, pl.ds(r0, ROWS_PER_TILE)])

  return pl.kernel(body, out_type=out_type, mesh=_MESH,
                   scratch_types=scratch)


def _make_sc_count():
  out_type = (jax.ShapeDtypeStruct((NC, N_ACC, D), jnp.float32),)
  scratch = (
      pltpu.VMEM_SHARED((N_ACC, D), jnp.float32),
      pltpu.VMEM((C,), jnp.int32),
      pltpu.VMEM((C, D), jnp.float32),
  )

  def body(dst_hbm, zeros_hbm, ones_hbm, out_cnt, cnt_sh, didx, ones_v):
    cid = lax.axis_index("c")
    sid = lax.axis_index("s")
    wid = cid * NS + sid
    r0 = sid * ROWS_PER_TILE

    pltpu.sync_copy(zeros_hbm.at[pl.ds(r0, ROWS_PER_TILE)],
                    cnt_sh.at[pl.ds(r0, ROWS_PER_TILE)])
    pltpu.sync_copy(ones_hbm, ones_v)
    plsc.subcore_barrier()

    e0 = wid * EDGES_PER_TILE

    def chunk(t, carry):
      off = pl.multiple_of(e0 + t * C, C)
      pltpu.sync_copy(dst_hbm.at[pl.ds(off, C)], didx)
      pltpu.sync_copy(ones_v, cnt_sh.at[didx], add=True)
      return carry

    lax.fori_loop(0, CHUNKS_PER_TILE, chunk, 0)
    plsc.subcore_barrier()

    pltpu.sync_copy(cnt_sh.at[pl.ds(r0, ROWS_PER_TILE)],
                    out_cnt.at[cid, pl.ds(r0, ROWS_PER_TILE)])

  return pl.kernel(body, out_type=out_type, mesh=_MESH,
                   scratch_types=scratch)


def _tc_mid(sums1, cnt, feat, w1l, b1, w1r, w2l, w2r, b2):
  def body(s_ref, c_ref, f_ref, w1l_ref, b1_ref, w1r_ref, w2l_ref,
           w2r_ref, b2_ref, y2_ref, r2_ref):
    s = s_ref[0, :, :] + s_ref[1, :, :]
    c = c_ref[0, :, 0:1] + c_ref[1, :, 0:1]
    agg = s / jnp.maximum(c, 1.0)
    x2 = (jnp.dot(agg, w1l_ref[...], preferred_element_type=jnp.float32)
          + b1_ref[...]
          + jnp.dot(f_ref[...], w1r_ref[...],
                    preferred_element_type=jnp.float32))
    x2 = jnp.where(x2 >= 0, x2, 0.01 * x2)
    y2_ref[...] = jnp.dot(x2, w2l_ref[...],
                          preferred_element_type=jnp.float32)
    r2_ref[...] = (jnp.dot(x2, w2r_ref[...],
                           preferred_element_type=jnp.float32)
                   + b2_ref[...])

  return pl.pallas_call(
      body,
      out_shape=(jax.ShapeDtypeStruct((N_ACC, D), jnp.float32),
                 jax.ShapeDtypeStruct((N_ACC, D), jnp.float32)),
  )(sums1, cnt, feat, w1l, b1, w1r, w2l, w2r, b2)


def _tc_out(sums2, cnt, r2):
  def body(s_ref, c_ref, r_ref, o_ref):
    s = s_ref[0, :, :] + s_ref[1, :, :]
    c = c_ref[0, :, 0:1] + c_ref[1, :, 0:1]
    o_ref[...] = s / jnp.maximum(c, 1.0) + r_ref[...]

  return pl.pallas_call(
      body,
      out_shape=jax.ShapeDtypeStruct((N_ACC, D), jnp.float32),
  )(sums2, cnt, r2)


def kernel(features, edges, edges2, edge_features, additional_feature,
           W1_l, b1, W1_r, W2_l, b2, W2_r):
  del edges, edge_features, additional_feature
  src = edges2[0]
  dst = edges2[1]
  pad = E_PAD - E
  src_p = jnp.concatenate([src, jnp.zeros((pad,), jnp.int32)])
  dst_p = jnp.concatenate([dst, jnp.full((pad,), DUMMY_DST, jnp.int32)])
  feat_p = jnp.pad(features, ((0, N_ACC - N), (0, 0)))
  zeros_big = jnp.zeros((N_ACC, D), jnp.float32)
  ones = jnp.ones((C, D), jnp.float32)

  sc_agg = _make_sc_agg()
  sc_count = _make_sc_count()

  (cnt,) = sc_count(dst_p, zeros_big, ones)
  (sums1,) = sc_agg(feat_p, src_p, dst_p, zeros_big)
  y2, r2 = _tc_mid(sums1, cnt, feat_p, W1_l.T, b1[None, :], W1_r.T,
                   W2_l.T, W2_r.T, b2[None, :])
  (sums2,) = sc_agg(y2, src_p, dst_p, zeros_big)
  out = _tc_out(sums2, cnt, r2)
  return out[:N]
